# baseline (speedup 1.0000x reference)
import functools

import jax
import jax.numpy as jnp
from jax import lax
from jax.experimental import pallas as pl
from jax.experimental.pallas import tpu as pltpu
from jax.experimental.pallas import tpu_sc as plsc

_NC = 2
_NS = 16
_NW = _NC * _NS


def _sc_gather_mean(idx, table):
    B, L = idx.shape
    _, D = table.shape
    bpw = B // _NW

    mesh = plsc.VectorSubcoreMesh(core_axis_name="c", subcore_axis_name="s")

    @functools.partial(
        pl.kernel,
        out_type=jax.ShapeDtypeStruct((B, D), jnp.float32),
        mesh=mesh,
        scratch_types=[
            pltpu.VMEM((bpw, L), jnp.int32),
            pltpu.VMEM((bpw, L, D), jnp.float32),
            pltpu.VMEM((bpw, D), jnp.float32),
            pltpu.SemaphoreType.DMA,
        ],
        compiler_params=pltpu.CompilerParams(use_tc_tiling_on_sc=False),
    )
    def gather_mean(idx_hbm, table_hbm, out_hbm, idx_v, rows_v, ctx_v, sem):
        wid = lax.axis_index("s") * _NC + lax.axis_index("c")
        base = wid * bpw
        pltpu.sync_copy(idx_hbm.at[pl.ds(base, bpw)], idx_v)
        copies = [
            pltpu.async_copy(table_hbm.at[idx_v.at[b]], rows_v.at[b], sem)
            for b in range(bpw)
        ]
        for c in copies:
            c.wait()

        scale = jnp.float32(1.0 / L)

        def pool_row(b, carry):
            for h in range(D // 16):
                acc = rows_v[b, 0, pl.ds(h * 16, 16)]
                for j in range(1, L):
                    acc = acc + rows_v[b, j, pl.ds(h * 16, 16)]
                ctx_v[b, pl.ds(h * 16, 16)] = acc * scale
            return carry

        lax.fori_loop(0, bpw, pool_row, 0)
        pltpu.sync_copy(ctx_v, out_hbm.at[pl.ds(base, bpw)])

    return gather_mean(idx, table)


def _tc_dense(ctx, W, b2d, bv):
    B, D = ctx.shape
    V = W.shape[1]

    def mm(ctx_ref, w_ref, b_ref, out_ref):
        out_ref[...] = (
            jnp.dot(ctx_ref[...], w_ref[...], preferred_element_type=jnp.float32)
            + b_ref[...]
        )

    return pl.pallas_call(
        mm,
        grid=(pl.cdiv(V, bv),),
        in_specs=[
            pl.BlockSpec((B, D), lambda i: (0, 0)),
            pl.BlockSpec((D, bv), lambda i: (0, i)),
            pl.BlockSpec((1, bv), lambda i: (0, i)),
        ],
        out_specs=pl.BlockSpec((B, bv), lambda i: (0, i)),
        out_shape=jax.ShapeDtypeStruct((B, V), jnp.float32),
        compiler_params=pltpu.CompilerParams(
            dimension_semantics=("arbitrary",),
        ),
    )(ctx, W, b2d)


def kernel(inputs, table, W, b):
    ctx = _sc_gather_mean(inputs.astype(jnp.int32), table)
    return _tc_dense(ctx, W, b.reshape(1, -1), 2048)
